# baseline (speedup 1.0000x reference)
import functools

import jax
import jax.numpy as jnp
from jax.experimental import pallas as pl
from jax.experimental.pallas import tpu as pltpu


def _chamfer_kernel(s_ref, t_ref, fwd_ref, bwd_ref, fmin_scr, bsum_scr):
    j = pl.program_id(1)
    nj = pl.num_programs(1)

    s = s_ref[0]
    t = t_ref[0]

    d2 = jax.lax.dot_general(
        s, t, (((1,), (1,)), ((), ())),
        preferred_element_type=jnp.float32,
        precision=jax.lax.Precision.HIGHEST,
    )

    tile_fmin = jnp.min(d2, axis=1, keepdims=True)
    tile_bsum = jnp.sum(jnp.min(d2, axis=0))

    @pl.when(j == 0)
    def _():
        fmin_scr[...] = tile_fmin
        bsum_scr[0] = tile_bsum

    @pl.when(j > 0)
    def _():
        fmin_scr[...] = jnp.minimum(fmin_scr[...], tile_fmin)
        bsum_scr[0] = bsum_scr[0] + tile_bsum

    @pl.when(j == nj - 1)
    def _():
        fwd_ref[...] = jnp.full(fwd_ref.shape, jnp.sum(fmin_scr[...]), jnp.float32)
        bwd_ref[...] = jnp.full(bwd_ref.shape, bsum_scr[0], jnp.float32)


@functools.partial(jax.jit, static_argnames=("tm",))
def _chamfer_sums(source_cloud, target_cloud, tm=512):
    B, N, _ = source_cloud.shape
    M = target_cloud.shape[1]
    nj = M // tm

    s = source_cloud[:, :, :3]
    t = target_cloud[:, :, :3]
    s_sq = jnp.sum(s * s, axis=2, keepdims=True)
    t_sq = jnp.sum(t * t, axis=2, keepdims=True)
    ones_s = jnp.ones((B, N, 1), jnp.float32)
    ones_t = jnp.ones((B, M, 1), jnp.float32)
    zeros_s = jnp.zeros((B, N, 3), jnp.float32)
    zeros_t = jnp.zeros((B, M, 3), jnp.float32)
    s_aug = jnp.concatenate([-2.0 * s, s_sq, ones_s, zeros_s], axis=2)
    t_aug = jnp.concatenate([t, ones_t, t_sq, zeros_t], axis=2)

    fwd, bwd = pl.pallas_call(
        _chamfer_kernel,
        grid=(B, nj),
        in_specs=[
            pl.BlockSpec((1, N, 8), lambda b, j: (b, 0, 0)),
            pl.BlockSpec((1, tm, 8), lambda b, j: (b, j, 0)),
        ],
        out_specs=[
            pl.BlockSpec((1, 8, 128), lambda b, j: (b, 0, 0)),
            pl.BlockSpec((1, 8, 128), lambda b, j: (b, 0, 0)),
        ],
        out_shape=[
            jax.ShapeDtypeStruct((B, 8, 128), jnp.float32),
            jax.ShapeDtypeStruct((B, 8, 128), jnp.float32),
        ],
        scratch_shapes=[
            pltpu.VMEM((N, 1), jnp.float32),
            pltpu.SMEM((1,), jnp.float32),
        ],
        compiler_params=pltpu.CompilerParams(
            dimension_semantics=("parallel", "arbitrary"),
        ),
    )(s_aug, t_aug)
    return fwd[:, 0, 0], bwd[:, 0, 0]


def kernel(source_cloud, target_cloud):
    G = source_cloud.shape[1]
    fwd_sums, bwd_sums = _chamfer_sums(source_cloud, target_cloud)
    return (fwd_sums.mean() + bwd_sums.mean()) / G


# speedup vs baseline: 2.3248x; 2.3248x over previous
import functools

import jax
import jax.numpy as jnp
from jax.experimental import pallas as pl
from jax.experimental.pallas import tpu as pltpu


def _chamfer_kernel(s2_ref, t_ref, fwd_ref, bwd_ref, fmin_scr, bsum_scr):
    j = pl.program_id(1)
    nj = pl.num_programs(1)

    s2 = s2_ref[0]
    t = t_ref[0]

    dot2 = jax.lax.dot_general(
        s2, t, (((1,), (1,)), ((), ())), preferred_element_type=jnp.float32
    )

    s_sq = 0.25 * jnp.sum(s2 * s2, axis=1, keepdims=True)
    t_sq = jnp.sum(t * t, axis=1, keepdims=True).T
    d2 = (s_sq + t_sq) + dot2

    tile_fmin = jnp.min(d2, axis=1, keepdims=True)
    tile_bsum = jnp.sum(jnp.min(d2, axis=0))

    @pl.when(j == 0)
    def _():
        fmin_scr[...] = tile_fmin
        bsum_scr[0] = tile_bsum

    @pl.when(j > 0)
    def _():
        fmin_scr[...] = jnp.minimum(fmin_scr[...], tile_fmin)
        bsum_scr[0] = bsum_scr[0] + tile_bsum

    @pl.when(j == nj - 1)
    def _():
        fwd_ref[...] = jnp.full(fwd_ref.shape, jnp.sum(fmin_scr[...]), jnp.float32)
        bwd_ref[...] = jnp.full(bwd_ref.shape, bsum_scr[0], jnp.float32)


@functools.partial(jax.jit, static_argnames=("tm",))
def _chamfer_sums(source_cloud, target_cloud, tm=512):
    B, N, _ = source_cloud.shape
    M = target_cloud.shape[1]
    nj = M // tm

    s2 = -2.0 * source_cloud[:, :, :3]
    t = target_cloud[:, :, :3]

    fwd, bwd = pl.pallas_call(
        _chamfer_kernel,
        grid=(B, nj),
        in_specs=[
            pl.BlockSpec((1, N, 3), lambda b, j: (b, 0, 0)),
            pl.BlockSpec((1, tm, 3), lambda b, j: (b, j, 0)),
        ],
        out_specs=[
            pl.BlockSpec((1, 8, 128), lambda b, j: (b, 0, 0)),
            pl.BlockSpec((1, 8, 128), lambda b, j: (b, 0, 0)),
        ],
        out_shape=[
            jax.ShapeDtypeStruct((B, 8, 128), jnp.float32),
            jax.ShapeDtypeStruct((B, 8, 128), jnp.float32),
        ],
        scratch_shapes=[
            pltpu.VMEM((N, 1), jnp.float32),
            pltpu.SMEM((1,), jnp.float32),
        ],
        compiler_params=pltpu.CompilerParams(
            dimension_semantics=("parallel", "arbitrary"),
        ),
    )(s2, t)
    return fwd[:, 0, 0], bwd[:, 0, 0]


def kernel(source_cloud, target_cloud):
    G = source_cloud.shape[1]
    fwd_sums, bwd_sums = _chamfer_sums(source_cloud, target_cloud)
    return (fwd_sums.mean() + bwd_sums.mean()) / G


# speedup vs baseline: 2.4036x; 1.0339x over previous
import functools

import jax
import jax.numpy as jnp
from jax.experimental import pallas as pl
from jax.experimental.pallas import tpu as pltpu


def _split3(x):
    h = x.astype(jnp.bfloat16).astype(jnp.float32)
    r = x - h
    m = r.astype(jnp.bfloat16).astype(jnp.float32)
    l = r - m
    return h, m, l


def _chamfer_kernel(s_ref, t_ref, fwd_ref, bwd_ref, fmin_scr, bsum_scr):
    j = pl.program_id(1)
    nj = pl.num_programs(1)

    s = s_ref[0]
    t = t_ref[0]

    d2 = jax.lax.dot_general(
        s, t, (((1,), (1,)), ((), ())), preferred_element_type=jnp.float32
    )

    tile_fmin = jnp.min(d2, axis=1, keepdims=True)
    tile_bsum = jnp.sum(jnp.min(d2, axis=0))

    @pl.when(j == 0)
    def _():
        fmin_scr[...] = tile_fmin
        bsum_scr[0] = tile_bsum

    @pl.when(j > 0)
    def _():
        fmin_scr[...] = jnp.minimum(fmin_scr[...], tile_fmin)
        bsum_scr[0] = bsum_scr[0] + tile_bsum

    @pl.when(j == nj - 1)
    def _():
        fwd_ref[...] = jnp.full(fwd_ref.shape, jnp.sum(fmin_scr[...]), jnp.float32)
        bwd_ref[...] = jnp.full(bwd_ref.shape, bsum_scr[0], jnp.float32)


@functools.partial(jax.jit, static_argnames=("tm",))
def _chamfer_sums(source_cloud, target_cloud, tm=512):
    B, N, _ = source_cloud.shape
    M = target_cloud.shape[1]
    nj = M // tm

    s = source_cloud[:, :, :3]
    t = target_cloud[:, :, :3]
    s_sq = jnp.sum(s * s, axis=2, keepdims=True)
    t_sq = jnp.sum(t * t, axis=2, keepdims=True)
    hs, ms, ls = _split3(s_sq)
    ht, mt, lt = _split3(t_sq)
    ones_s = jnp.ones((B, N, 1), jnp.float32)
    ones_t = jnp.ones((B, M, 1), jnp.float32)
    zeros_s = jnp.zeros((B, N, 7), jnp.float32)
    zeros_t = jnp.zeros((B, M, 7), jnp.float32)
    s_aug = jnp.concatenate(
        [-2.0 * s, hs, ms, ls, ones_s, ones_s, ones_s, zeros_s], axis=2
    )
    t_aug = jnp.concatenate(
        [t, ones_t, ones_t, ones_t, ht, mt, lt, zeros_t], axis=2
    )

    fwd, bwd = pl.pallas_call(
        _chamfer_kernel,
        grid=(B, nj),
        in_specs=[
            pl.BlockSpec((1, N, 16), lambda b, j: (b, 0, 0)),
            pl.BlockSpec((1, tm, 16), lambda b, j: (b, j, 0)),
        ],
        out_specs=[
            pl.BlockSpec((1, 8, 128), lambda b, j: (b, 0, 0)),
            pl.BlockSpec((1, 8, 128), lambda b, j: (b, 0, 0)),
        ],
        out_shape=[
            jax.ShapeDtypeStruct((B, 8, 128), jnp.float32),
            jax.ShapeDtypeStruct((B, 8, 128), jnp.float32),
        ],
        scratch_shapes=[
            pltpu.VMEM((N, 1), jnp.float32),
            pltpu.SMEM((1,), jnp.float32),
        ],
        compiler_params=pltpu.CompilerParams(
            dimension_semantics=("parallel", "arbitrary"),
        ),
    )(s_aug, t_aug)
    return fwd[:, 0, 0], bwd[:, 0, 0]


def kernel(source_cloud, target_cloud):
    G = source_cloud.shape[1]
    fwd_sums, bwd_sums = _chamfer_sums(source_cloud, target_cloud)
    return (fwd_sums.mean() + bwd_sums.mean()) / G


# speedup vs baseline: 4.2599x; 1.7723x over previous
import functools

import jax
import jax.numpy as jnp
from jax.experimental import pallas as pl
from jax.experimental.pallas import tpu as pltpu


def _split3(x):
    h = x.astype(jnp.bfloat16).astype(jnp.float32)
    r = x - h
    m = r.astype(jnp.bfloat16).astype(jnp.float32)
    l = r - m
    return h, m, l


def _fill_aug(aug_ref, pts, coord_scale, norm_off, ones_off):
    n = pts.shape[0]
    sq = jnp.sum(pts * pts, axis=1, keepdims=True)
    h, m, l = _split3(sq)
    aug_ref[:, 0:3] = coord_scale * pts
    aug_ref[:, norm_off : norm_off + 1] = h
    aug_ref[:, norm_off + 1 : norm_off + 2] = m
    aug_ref[:, norm_off + 2 : norm_off + 3] = l
    aug_ref[:, ones_off : ones_off + 3] = jnp.ones((n, 3), jnp.float32)
    aug_ref[:, 9:16] = jnp.zeros((n, 7), jnp.float32)


def _chamfer_kernel(
    s_ref, t_ref, fwd_ref, bwd_ref, saug_scr, taug_scr, fmin_scr, bsum_scr
):
    j = pl.program_id(1)
    nj = pl.num_programs(1)

    @pl.when(j == 0)
    def _():
        _fill_aug(saug_scr, s_ref[0], -2.0, norm_off=3, ones_off=6)

    _fill_aug(taug_scr, t_ref[0], 1.0, norm_off=6, ones_off=3)

    d2 = jax.lax.dot_general(
        saug_scr[...],
        taug_scr[...],
        (((1,), (1,)), ((), ())),
        preferred_element_type=jnp.float32,
    )

    tile_fmin = jnp.min(d2, axis=1, keepdims=True)
    tile_bsum = jnp.sum(jnp.min(d2, axis=0))

    @pl.when(j == 0)
    def _():
        fmin_scr[...] = tile_fmin
        bsum_scr[0] = tile_bsum

    @pl.when(j > 0)
    def _():
        fmin_scr[...] = jnp.minimum(fmin_scr[...], tile_fmin)
        bsum_scr[0] = bsum_scr[0] + tile_bsum

    @pl.when(j == nj - 1)
    def _():
        fwd_ref[...] = jnp.full(fwd_ref.shape, jnp.sum(fmin_scr[...]), jnp.float32)
        bwd_ref[...] = jnp.full(bwd_ref.shape, bsum_scr[0], jnp.float32)


@functools.partial(jax.jit, static_argnames=("tm",))
def _chamfer_sums(source_cloud, target_cloud, tm=512):
    B, N, _ = source_cloud.shape
    M = target_cloud.shape[1]
    nj = M // tm

    fwd, bwd = pl.pallas_call(
        _chamfer_kernel,
        grid=(B, nj),
        in_specs=[
            pl.BlockSpec((1, N, 3), lambda b, j: (b, 0, 0)),
            pl.BlockSpec((1, tm, 3), lambda b, j: (b, j, 0)),
        ],
        out_specs=[
            pl.BlockSpec((1, 8, 128), lambda b, j: (b, 0, 0)),
            pl.BlockSpec((1, 8, 128), lambda b, j: (b, 0, 0)),
        ],
        out_shape=[
            jax.ShapeDtypeStruct((B, 8, 128), jnp.float32),
            jax.ShapeDtypeStruct((B, 8, 128), jnp.float32),
        ],
        scratch_shapes=[
            pltpu.VMEM((N, 16), jnp.float32),
            pltpu.VMEM((tm, 16), jnp.float32),
            pltpu.VMEM((N, 1), jnp.float32),
            pltpu.SMEM((1,), jnp.float32),
        ],
        compiler_params=pltpu.CompilerParams(
            dimension_semantics=("parallel", "arbitrary"),
        ),
    )(source_cloud[:, :, :3], target_cloud[:, :, :3])
    return fwd[:, 0, 0], bwd[:, 0, 0]


def kernel(source_cloud, target_cloud):
    G = source_cloud.shape[1]
    fwd_sums, bwd_sums = _chamfer_sums(source_cloud, target_cloud)
    return (fwd_sums.mean() + bwd_sums.mean()) / G


# speedup vs baseline: 4.8299x; 1.1338x over previous
import functools

import jax
import jax.numpy as jnp
from jax.experimental import pallas as pl
from jax.experimental.pallas import tpu as pltpu


def _split3(x):
    h = x.astype(jnp.bfloat16).astype(jnp.float32)
    r = x - h
    m = r.astype(jnp.bfloat16).astype(jnp.float32)
    l = r - m
    return h, m, l


def _fill_aug(aug_ref, pts, coord_scale, norm_off, ones_off):
    n = pts.shape[0]
    sq = jnp.sum(pts * pts, axis=1, keepdims=True)
    h, m, l = _split3(sq)
    aug_ref[:, 0:3] = coord_scale * pts
    aug_ref[:, norm_off : norm_off + 1] = h
    aug_ref[:, norm_off + 1 : norm_off + 2] = m
    aug_ref[:, norm_off + 2 : norm_off + 3] = l
    aug_ref[:, ones_off : ones_off + 3] = jnp.ones((n, 3), jnp.float32)
    aug_ref[:, 9:16] = jnp.zeros((n, 7), jnp.float32)


def _chamfer_kernel(
    s_ref, t_ref, fwd_ref, bwd_ref, saug_scr, taug_scr, fmin_scr, bsum_scr, *, ch
):
    j = pl.program_id(1)
    nj = pl.num_programs(1)

    @pl.when(j == 0)
    def _():
        _fill_aug(saug_scr, s_ref[0], -2.0, norm_off=3, ones_off=6)

    _fill_aug(taug_scr, t_ref[0], 1.0, norm_off=6, ones_off=3)

    tm = taug_scr.shape[0]
    saug = saug_scr[...]
    fmins = []
    bsums = []
    for c in range(tm // ch):
        d2 = jax.lax.dot_general(
            saug,
            taug_scr[c * ch : (c + 1) * ch, :],
            (((1,), (1,)), ((), ())),
            preferred_element_type=jnp.float32,
        )
        fmins.append(jnp.min(d2, axis=1, keepdims=True))
        bsums.append(jnp.sum(jnp.min(d2, axis=0)))

    tile_fmin = fmins[0]
    for fm in fmins[1:]:
        tile_fmin = jnp.minimum(tile_fmin, fm)
    tile_bsum = sum(bsums)

    @pl.when(j == 0)
    def _():
        fmin_scr[...] = tile_fmin
        bsum_scr[0] = tile_bsum

    @pl.when(j > 0)
    def _():
        fmin_scr[...] = jnp.minimum(fmin_scr[...], tile_fmin)
        bsum_scr[0] = bsum_scr[0] + tile_bsum

    @pl.when(j == nj - 1)
    def _():
        fwd_ref[...] = jnp.full(fwd_ref.shape, jnp.sum(fmin_scr[...]), jnp.float32)
        bwd_ref[...] = jnp.full(bwd_ref.shape, bsum_scr[0], jnp.float32)


@functools.partial(jax.jit, static_argnames=("tm", "ch"))
def _chamfer_sums(source_cloud, target_cloud, tm=1024, ch=256):
    B, N, _ = source_cloud.shape
    M = target_cloud.shape[1]
    nj = M // tm

    fwd, bwd = pl.pallas_call(
        functools.partial(_chamfer_kernel, ch=ch),
        grid=(B, nj),
        in_specs=[
            pl.BlockSpec((1, N, 3), lambda b, j: (b, 0, 0)),
            pl.BlockSpec((1, tm, 3), lambda b, j: (b, j, 0)),
        ],
        out_specs=[
            pl.BlockSpec((1, 8, 128), lambda b, j: (b, 0, 0)),
            pl.BlockSpec((1, 8, 128), lambda b, j: (b, 0, 0)),
        ],
        out_shape=[
            jax.ShapeDtypeStruct((B, 8, 128), jnp.float32),
            jax.ShapeDtypeStruct((B, 8, 128), jnp.float32),
        ],
        scratch_shapes=[
            pltpu.VMEM((N, 16), jnp.float32),
            pltpu.VMEM((tm, 16), jnp.float32),
            pltpu.VMEM((N, 1), jnp.float32),
            pltpu.SMEM((1,), jnp.float32),
        ],
        compiler_params=pltpu.CompilerParams(
            dimension_semantics=("parallel", "arbitrary"),
        ),
    )(source_cloud[:, :, :3], target_cloud[:, :, :3])
    return fwd[:, 0, 0], bwd[:, 0, 0]


def kernel(source_cloud, target_cloud):
    G = source_cloud.shape[1]
    fwd_sums, bwd_sums = _chamfer_sums(source_cloud, target_cloud)
    return (fwd_sums.mean() + bwd_sums.mean()) / G


# speedup vs baseline: 5.1598x; 1.0683x over previous
import functools

import jax
import jax.numpy as jnp
from jax.experimental import pallas as pl
from jax.experimental.pallas import tpu as pltpu


def _split3(x):
    h = x.astype(jnp.bfloat16).astype(jnp.float32)
    r = x - h
    m = r.astype(jnp.bfloat16).astype(jnp.float32)
    l = r - m
    return h, m, l


def _fill_aug(aug_ref, pts, coord_scale, norm_off, ones_off):
    n = pts.shape[0]
    sq = jnp.sum(pts * pts, axis=1, keepdims=True)
    h, m, l = _split3(sq)
    aug_ref[:, 0:3] = coord_scale * pts
    aug_ref[:, norm_off : norm_off + 1] = h
    aug_ref[:, norm_off + 1 : norm_off + 2] = m
    aug_ref[:, norm_off + 2 : norm_off + 3] = l
    aug_ref[:, ones_off : ones_off + 3] = jnp.ones((n, 3), jnp.float32)
    aug_ref[:, 9:16] = jnp.zeros((n, 7), jnp.float32)


def _chamfer_kernel(
    s_ref, t_ref, fwd_ref, bwd_ref, saug_scr, taug_scr, fmin_scr, bsum_scr, *, ch
):
    j = pl.program_id(1)
    nj = pl.num_programs(1)

    @pl.when(j == 0)
    def _():
        _fill_aug(saug_scr, s_ref[0], -2.0, norm_off=3, ones_off=6)

    _fill_aug(taug_scr, t_ref[0], 1.0, norm_off=6, ones_off=3)

    tm = taug_scr.shape[0]
    saug = saug_scr[...]
    fmins = []
    bsums = []
    for c in range(tm // ch):
        d2 = jax.lax.dot_general(
            saug,
            taug_scr[c * ch : (c + 1) * ch, :],
            (((1,), (1,)), ((), ())),
            preferred_element_type=jnp.float32,
        )
        fmins.append(jnp.min(d2, axis=1, keepdims=True))
        bsums.append(jnp.sum(jnp.min(d2, axis=0)))

    tile_fmin = fmins[0]
    for fm in fmins[1:]:
        tile_fmin = jnp.minimum(tile_fmin, fm)
    tile_bsum = sum(bsums)

    @pl.when(j == 0)
    def _():
        fmin_scr[...] = tile_fmin
        bsum_scr[0] = tile_bsum

    @pl.when(j > 0)
    def _():
        fmin_scr[...] = jnp.minimum(fmin_scr[...], tile_fmin)
        bsum_scr[0] = bsum_scr[0] + tile_bsum

    @pl.when(j == nj - 1)
    def _():
        fwd_ref[...] = jnp.full(fwd_ref.shape, jnp.sum(fmin_scr[...]), jnp.float32)
        bwd_ref[...] = jnp.full(bwd_ref.shape, bsum_scr[0], jnp.float32)


@functools.partial(jax.jit, static_argnames=("tm", "ch"))
def _chamfer_sums(source_cloud, target_cloud, tm=2048, ch=256):
    B, N, _ = source_cloud.shape
    M = target_cloud.shape[1]
    nj = M // tm

    fwd, bwd = pl.pallas_call(
        functools.partial(_chamfer_kernel, ch=ch),
        grid=(B, nj),
        in_specs=[
            pl.BlockSpec((1, N, 3), lambda b, j: (b, 0, 0)),
            pl.BlockSpec((1, tm, 3), lambda b, j: (b, j, 0)),
        ],
        out_specs=[
            pl.BlockSpec((1, 8, 128), lambda b, j: (b, 0, 0)),
            pl.BlockSpec((1, 8, 128), lambda b, j: (b, 0, 0)),
        ],
        out_shape=[
            jax.ShapeDtypeStruct((B, 8, 128), jnp.float32),
            jax.ShapeDtypeStruct((B, 8, 128), jnp.float32),
        ],
        scratch_shapes=[
            pltpu.VMEM((N, 16), jnp.float32),
            pltpu.VMEM((tm, 16), jnp.float32),
            pltpu.VMEM((N, 1), jnp.float32),
            pltpu.SMEM((1,), jnp.float32),
        ],
        compiler_params=pltpu.CompilerParams(
            dimension_semantics=("parallel", "arbitrary"),
        ),
    )(source_cloud[:, :, :3], target_cloud[:, :, :3])
    return fwd[:, 0, 0], bwd[:, 0, 0]


def kernel(source_cloud, target_cloud):
    G = source_cloud.shape[1]
    fwd_sums, bwd_sums = _chamfer_sums(source_cloud, target_cloud)
    return (fwd_sums.mean() + bwd_sums.mean()) / G
